# Initial kernel scaffold; baseline (speedup 1.0000x reference)
import jax
import jax.numpy as jnp
from jax.experimental import pallas as pl


def kernel(unary, binary, index1, index2):
    raise NotImplementedError("write your pallas kernel here")


# speedup vs baseline: 2.0745x; 2.0745x over previous
import jax
import jax.numpy as jnp
from jax import lax
from jax.experimental import pallas as pl
from jax.experimental.pallas import tpu as pltpu
from jax.experimental.pallas import tpu_sc as plsc

E = 320000
V = 10000
D = 128
F = 16
NW = 32
PER_W = E // NW
CHUNK = 80
NCH = PER_W // CHUNK


def _join_body(unary, binary, idx1, idx2, out, idx1_v, idx2_v, rows1, rows2,
               bin_v):
    c = lax.axis_index("c")
    s = lax.axis_index("s")
    wid = s * 2 + c
    w0 = pl.multiple_of(wid * PER_W, 8)
    pltpu.sync_copy(idx1.at[pl.ds(w0, PER_W)], idx1_v)
    pltpu.sync_copy(idx2.at[pl.ds(w0, PER_W)], idx2_v)

    def chunk(i, carry):
        base = pl.multiple_of(i * CHUNK, 8)
        g = pl.multiple_of(w0 + base, 8)
        pltpu.sync_copy(unary.at[idx1_v.at[pl.ds(base, CHUNK)]], rows1)
        pltpu.sync_copy(rows1, out.at[pl.ds(g, CHUNK), pl.ds(0, D)])
        pltpu.sync_copy(unary.at[idx2_v.at[pl.ds(base, CHUNK)]], rows2)
        pltpu.sync_copy(rows2, out.at[pl.ds(g, CHUNK), pl.ds(D, D)])
        pltpu.sync_copy(binary.at[pl.ds(g, CHUNK), :], bin_v)
        pltpu.sync_copy(bin_v, out.at[pl.ds(g, CHUNK), pl.ds(2 * D, F)])
        return carry

    lax.fori_loop(0, NCH, chunk, 0)


def kernel(unary, binary, index1, index2):
    mesh = plsc.VectorSubcoreMesh(core_axis_name="c", subcore_axis_name="s")
    f = pl.kernel(
        _join_body,
        mesh=mesh,
        out_type=jax.ShapeDtypeStruct((E, 2 * D + F), jnp.float32),
        scratch_types=[
            pltpu.VMEM((PER_W,), jnp.int32),
            pltpu.VMEM((PER_W,), jnp.int32),
            pltpu.VMEM((CHUNK, D), jnp.float32),
            pltpu.VMEM((CHUNK, D), jnp.float32),
            pltpu.VMEM((CHUNK, F), jnp.float32),
        ],
    )
    return f(unary, binary, index1.astype(jnp.int32), index2.astype(jnp.int32))


# speedup vs baseline: 2.8201x; 1.3594x over previous
import jax
import jax.numpy as jnp
from jax import lax
from jax.experimental import pallas as pl
from jax.experimental.pallas import tpu as pltpu
from jax.experimental.pallas import tpu_sc as plsc

E = 320000
V = 10000
D = 128
F = 16
NW = 32
PER_W = E // NW
CHUNK = 80
NCH = PER_W // CHUNK


def _join_body(unary, binary, idx1, idx2, out, idx1_v, idx2_v, rows1, rows2,
               bin_v, gsem, ssem):
    c = lax.axis_index("c")
    s = lax.axis_index("s")
    wid = s * 2 + c
    w0 = pl.multiple_of(wid * PER_W, 8)
    pltpu.sync_copy(idx1.at[pl.ds(w0, PER_W)], idx1_v)
    pltpu.sync_copy(idx2.at[pl.ds(w0, PER_W)], idx2_v)

    def start_in(slot, i):
        base = pl.multiple_of(i * CHUNK, 8)
        g = pl.multiple_of(w0 + base, 8)
        pltpu.async_copy(unary.at[idx1_v.at[pl.ds(base, CHUNK)]],
                         rows1.at[slot], gsem.at[slot])
        pltpu.async_copy(unary.at[idx2_v.at[pl.ds(base, CHUNK)]],
                         rows2.at[slot], gsem.at[slot])
        pltpu.async_copy(binary.at[pl.ds(g, CHUNK), :], bin_v.at[slot],
                         gsem.at[slot])

    def wait_in(slot):
        pltpu.make_async_copy(unary.at[idx1_v.at[pl.ds(0, CHUNK)]],
                              rows1.at[slot], gsem.at[slot]).wait()
        pltpu.make_async_copy(unary.at[idx2_v.at[pl.ds(0, CHUNK)]],
                              rows2.at[slot], gsem.at[slot]).wait()
        pltpu.make_async_copy(binary.at[pl.ds(0, CHUNK), :], bin_v.at[slot],
                              gsem.at[slot]).wait()

    def start_out(slot, i):
        g = pl.multiple_of(w0 + i * CHUNK, 8)
        pltpu.async_copy(rows1.at[slot], out.at[pl.ds(g, CHUNK), pl.ds(0, D)],
                         ssem.at[slot])
        pltpu.async_copy(rows2.at[slot], out.at[pl.ds(g, CHUNK), pl.ds(D, D)],
                         ssem.at[slot])
        pltpu.async_copy(bin_v.at[slot],
                         out.at[pl.ds(g, CHUNK), pl.ds(2 * D, F)],
                         ssem.at[slot])

    def wait_out(slot):
        pltpu.make_async_copy(rows1.at[slot],
                              out.at[pl.ds(w0, CHUNK), pl.ds(0, D)],
                              ssem.at[slot]).wait()
        pltpu.make_async_copy(rows2.at[slot],
                              out.at[pl.ds(w0, CHUNK), pl.ds(D, D)],
                              ssem.at[slot]).wait()
        pltpu.make_async_copy(bin_v.at[slot],
                              out.at[pl.ds(w0, CHUNK), pl.ds(2 * D, F)],
                              ssem.at[slot]).wait()

    start_in(0, 0)

    def body(i, carry):
        slot = lax.rem(i, 2)
        nslot = 1 - slot

        @pl.when(i + 1 < NCH)
        def _():
            @pl.when(i >= 1)
            def _():
                wait_out(nslot)
            start_in(nslot, i + 1)

        wait_in(slot)
        start_out(slot, i)
        return carry

    lax.fori_loop(0, NCH, body, 0)
    wait_out((NCH - 2) % 2)
    wait_out((NCH - 1) % 2)


def kernel(unary, binary, index1, index2):
    mesh = plsc.VectorSubcoreMesh(core_axis_name="c", subcore_axis_name="s")
    f = pl.kernel(
        _join_body,
        mesh=mesh,
        out_type=jax.ShapeDtypeStruct((E, 2 * D + F), jnp.float32),
        scratch_types=[
            pltpu.VMEM((PER_W,), jnp.int32),
            pltpu.VMEM((PER_W,), jnp.int32),
            pltpu.VMEM((2, CHUNK, D), jnp.float32),
            pltpu.VMEM((2, CHUNK, D), jnp.float32),
            pltpu.VMEM((2, CHUNK, F), jnp.float32),
            pltpu.SemaphoreType.DMA((2,)),
            pltpu.SemaphoreType.DMA((2,)),
        ],
    )
    return f(unary, binary, index1.astype(jnp.int32), index2.astype(jnp.int32))
